# baseline (speedup 1.0000x reference)
import functools

import jax
import jax.numpy as jnp
from jax import lax
from jax.experimental import pallas as pl
from jax.experimental.pallas import tpu as pltpu
from jax.experimental.pallas import tpu_sc as plsc

N = 10000
E = 320000
T = 12
D = 128
H = 32
P = 12
OUT = 1

NC = 2
NS = 16
NW = NC * NS

NP = 10112
RPT = NP // NS
CHUNK = 128
CPT = 80
EPT = CPT * CHUNK
EP = NW * EPT

NB = 8
BR = NP // NB


def _make_agg(w, nbuf):
    mesh = plsc.VectorSubcoreMesh(core_axis_name="c", subcore_axis_name="s", num_cores=NC, num_subcores=NS)

    @functools.partial(
        pl.kernel,
        out_type=jax.ShapeDtypeStruct((NC, NP, w), jnp.float32),
        mesh=mesh,
        scratch_types=(
            [pltpu.VMEM((CPT, CHUNK), jnp.int32),
             pltpu.VMEM((CPT, CHUNK), jnp.int32)]
            + [pltpu.VMEM((CHUNK, w), jnp.float32) for _ in range(nbuf)]
            + [pltpu.SemaphoreType.DMA for _ in range(nbuf)]
            + [pltpu.VMEM_SHARED((NP, w), jnp.float32)]
        ),
        compiler_params=pltpu.CompilerParams(use_tc_tiling_on_sc=False),
    )
    def agg(v_hbm, srcw_hbm, dstw_hbm, out_hbm, src_v, dst_v, *rest):
        bufs = rest[:nbuf]
        sems = rest[nbuf:2 * nbuf]
        acc_sh = rest[2 * nbuf]
        c = lax.axis_index("c")
        s = lax.axis_index("s")
        wid = c * NS + s

        def zbody(i, carry):
            for hh in range(w // 16):
                bufs[0][i, hh * 16:(hh + 1) * 16] = jnp.zeros((16,),
                                                              jnp.float32)
            return carry

        lax.fori_loop(0, CHUNK, zbody, 0)
        row0 = s * RPT
        off = 0
        for sz in (128, 128, 128, 128, 120):
            pltpu.sync_copy(bufs[0].at[pl.ds(0, sz)],
                            acc_sh.at[pl.ds(row0 + off, sz)])
            off += sz

        pltpu.sync_copy(srcw_hbm.at[wid], src_v)
        pltpu.sync_copy(dstw_hbm.at[wid], dst_v)
        for b in range(nbuf):
            pltpu.async_copy(v_hbm.at[src_v.at[b]], bufs[b], sems[b])
        plsc.subcore_barrier()

        def body(g, carry):
            for b in range(nbuf):
                j = g * nbuf + b
                pltpu.make_async_copy(v_hbm.at[src_v.at[j]],
                                      bufs[b], sems[b]).wait()
                pltpu.sync_copy(bufs[b], acc_sh.at[dst_v.at[j]], add=True)
                nj = j + nbuf

                @pl.when(nj < CPT)
                def _():
                    pltpu.async_copy(v_hbm.at[src_v.at[nj]],
                                     bufs[b], sems[b])
            return carry

        lax.fori_loop(0, CPT // nbuf, body, 0)
        plsc.subcore_barrier()
        pltpu.sync_copy(acc_sh.at[pl.ds(row0, RPT)],
                        out_hbm.at[c, pl.ds(row0, RPT)])

    return agg


_agg32 = _make_agg(H, 8)
_agg64 = _make_agg(D // 2, 4)


def _xterm_kernel(o1_ref, ag_ref, w_ref, o_ref):
    xa = o1_ref[...] + ag_ref[0] + ag_ref[1]
    o_ref[...] = jnp.dot(xa, w_ref[...], preferred_element_type=jnp.float32)


def _xterm1_kernel(x_ref, aglo_ref, aghi_ref, w_ref, o_ref):
    hw = D // 2
    xlo = x_ref[:, :hw] + aglo_ref[0] + aglo_ref[1]
    xhi = x_ref[:, hw:] + aghi_ref[0] + aghi_ref[1]
    o_ref[...] = (jnp.dot(xlo, w_ref[:hw],
                          preferred_element_type=jnp.float32) +
                  jnp.dot(xhi, w_ref[hw:],
                          preferred_element_type=jnp.float32))


def _gate_a_kernel(xterm_ref, h_ref, aggh_ref, whzr_ref, bzr_ref,
                   z_ref, g_ref, xc_ref):
    h = h_ref[...]
    ha = h + aggh_ref[0] + aggh_ref[1]
    hz = jnp.dot(ha, whzr_ref[...],
                 preferred_element_type=jnp.float32) + bzr_ref[...]
    xt = xterm_ref[...]
    z = jax.nn.sigmoid(xt[:, :H] + hz[:, :H])
    r = jax.nn.sigmoid(xt[:, H:2 * H] + hz[:, H:])
    z_ref[...] = z
    g_ref[...] = r * h
    xc_ref[...] = xt[:, 2 * H:]


def _gate_b_kernel(z_ref, h_ref, g_ref, xc_ref, aggg_ref, whh_ref, bh_ref,
                   hn_ref):
    ga = g_ref[...] + aggg_ref[0] + aggg_ref[1]
    c = jnp.tanh(xc_ref[...] +
                 jnp.dot(ga, whh_ref[...], preferred_element_type=jnp.float32)
                 + bh_ref[...])
    z = z_ref[...]
    hn_ref[...] = z * h_ref[...] + (1.0 - z) * c


def _head_kernel(h_ref, w_ref, b_ref, o_ref):
    o_ref[...] = jax.nn.relu(
        jnp.dot(h_ref[...], w_ref[...], preferred_element_type=jnp.float32)
        + b_ref[...])


def _tc(body, out_shape, grid=None, in_specs=None, out_specs=None):
    kwargs = {}
    if grid is not None:
        kwargs = dict(grid=grid, in_specs=in_specs, out_specs=out_specs)
    return pl.pallas_call(body, out_shape=out_shape, **kwargs)


def kernel(x, edge_index, params):
    p1, p2 = params['l1'], params['l2']

    Wx1 = jnp.concatenate([p1['Wxz'], p1['Wxr'], p1['Wxh']], axis=1)
    Wx2 = jnp.concatenate([p2['Wxz'], p2['Wxr'], p2['Wxh']], axis=1)
    Whzr1 = jnp.concatenate([p1['Whz'], p1['Whr']], axis=1)
    Whzr2 = jnp.concatenate([p2['Whz'], p2['Whr']], axis=1)
    bzr1 = jnp.concatenate([p1['bz'], p1['br']]).reshape(1, 2 * H)
    bzr2 = jnp.concatenate([p2['bz'], p2['br']]).reshape(1, 2 * H)
    bh1 = p1['bh'].reshape(1, H)
    bh2 = p2['bh'].reshape(1, H)
    bfc = params['bfc'].reshape(1, P * OUT)

    pad = EP - E
    srcp = jnp.concatenate([edge_index[0],
                            jnp.zeros((pad,), jnp.int32)])
    dstp = jnp.concatenate([edge_index[1],
                            jnp.full((pad,), NP - 1, jnp.int32)])
    srcw = srcp.reshape(NW, CPT, CHUNK)
    dstw = dstp.reshape(NW, CPT, CHUNK)

    x_pad = jnp.pad(x, ((0, 0), (0, NP - N), (0, 0)))

    def xterm_tc(v_t, ag_t, Wx):
        return _tc(
            _xterm_kernel,
            jax.ShapeDtypeStruct((NP, 3 * H), jnp.float32),
        )(v_t, ag_t, Wx)

    def gate_a(xterm_t, h, aggh, Whzr, bzr):
        return _tc(
            _gate_a_kernel,
            (jax.ShapeDtypeStruct((NP, H), jnp.float32),
             jax.ShapeDtypeStruct((NP, H), jnp.float32),
             jax.ShapeDtypeStruct((NP, H), jnp.float32)),
            grid=(NB,),
            in_specs=[pl.BlockSpec((BR, 3 * H), lambda i: (i, 0)),
                      pl.BlockSpec((BR, H), lambda i: (i, 0)),
                      pl.BlockSpec((NC, BR, H), lambda i: (0, i, 0)),
                      pl.BlockSpec((H, 2 * H), lambda i: (0, 0)),
                      pl.BlockSpec((1, 2 * H), lambda i: (0, 0))],
            out_specs=[pl.BlockSpec((BR, H), lambda i: (i, 0))] * 3,
        )(xterm_t, h, aggh, Whzr, bzr)

    def gate_b(z, h, g, xc, aggg, Whh, bh):
        return _tc(
            _gate_b_kernel,
            jax.ShapeDtypeStruct((NP, H), jnp.float32),
            grid=(NB,),
            in_specs=[pl.BlockSpec((BR, H), lambda i: (i, 0)),
                      pl.BlockSpec((BR, H), lambda i: (i, 0)),
                      pl.BlockSpec((BR, H), lambda i: (i, 0)),
                      pl.BlockSpec((BR, H), lambda i: (i, 0)),
                      pl.BlockSpec((NC, BR, H), lambda i: (0, i, 0)),
                      pl.BlockSpec((H, H), lambda i: (0, 0)),
                      pl.BlockSpec((1, H), lambda i: (0, 0))],
            out_specs=pl.BlockSpec((BR, H), lambda i: (i, 0)),
        )(z, h, g, xc, aggg, Whh, bh)

    xterm1 = []
    for t in range(T):
        aglo_t = _agg64(x_pad[t, :, :D // 2], srcw, dstw)
        aghi_t = _agg64(x_pad[t, :, D // 2:], srcw, dstw)
        xterm1.append(_tc(
            _xterm1_kernel,
            jax.ShapeDtypeStruct((NP, 3 * H), jnp.float32),
        )(x_pad[t], aglo_t, aghi_t, Wx1))

    def run_layer(xterms, h, Whzr, bzr, Whh, bh):
        outs = []
        for t in range(T):
            aggh = _agg32(h, srcw, dstw)
            z, g, xc = gate_a(xterms[t], h, aggh, Whzr, bzr)
            aggg = _agg32(g, srcw, dstw)
            h = gate_b(z, h, g, xc, aggg, Whh, bh)
            outs.append(h)
        return h, outs

    h0 = jnp.zeros((NP, H), jnp.float32)
    h1, out1 = run_layer(xterm1, h0, Whzr1, bzr1, p1['Whh'], bh1)

    xterm2 = []
    for t in range(T):
        ago_t = _agg32(out1[t], srcw, dstw)
        xterm2.append(xterm_tc(out1[t], ago_t, Wx2))

    h2, _ = run_layer(xterm2, h1, Whzr2, bzr2, p2['Whh'], bh2)

    y = _tc(
        _head_kernel,
        jax.ShapeDtypeStruct((NP, P * OUT), jnp.float32),
    )(h2, params['Wfc'], bfc)

    y = y[:N].reshape(N, P, OUT)
    return jnp.swapaxes(y, 0, 1)


# speedup vs baseline: 1.3808x; 1.3808x over previous
import functools

import jax
import jax.numpy as jnp
from jax import lax
from jax.experimental import pallas as pl
from jax.experimental.pallas import tpu as pltpu
from jax.experimental.pallas import tpu_sc as plsc

N = 10000
E = 320000
T = 12
D = 128
H = 32
P = 12
OUT = 1

NC = 2
NS = 16
NW = NC * NS

NP = 10112
RPT = NP // NS
CHUNK = 128
CPT = 80
EPT = CPT * CHUNK
EP = NW * EPT

NB = 8
BR = NP // NB


def _make_agg(w, nbuf, stage_table):
    mesh = plsc.VectorSubcoreMesh(core_axis_name="c", subcore_axis_name="s",
                                  num_cores=NC, num_subcores=NS)

    scratch = (
        [pltpu.VMEM((CPT, CHUNK), jnp.int32),
         pltpu.VMEM((CPT, CHUNK), jnp.int32)]
        + [pltpu.VMEM((CHUNK, w), jnp.float32) for _ in range(nbuf)]
        + [pltpu.SemaphoreType.DMA for _ in range(2 * nbuf)]
        + [pltpu.VMEM_SHARED((NP, w), jnp.float32)]
        + ([pltpu.VMEM_SHARED((NP, w), jnp.float32)] if stage_table else [])
    )

    @functools.partial(
        pl.kernel,
        out_type=jax.ShapeDtypeStruct((NC, NP, w), jnp.float32),
        mesh=mesh,
        scratch_types=scratch,
        compiler_params=pltpu.CompilerParams(use_tc_tiling_on_sc=False),
    )
    def agg(v_hbm, srcw_hbm, dstw_hbm, out_hbm, src_v, dst_v, *rest):
        bufs = rest[:nbuf]
        gsems = rest[nbuf:2 * nbuf]
        ssems = rest[2 * nbuf:3 * nbuf]
        acc_sh = rest[3 * nbuf]
        table = rest[3 * nbuf + 1] if stage_table else v_hbm
        c = lax.axis_index("c")
        s = lax.axis_index("s")
        wid = c * NS + s

        def zbody(i, carry):
            for hh in range(w // 16):
                bufs[0][i, hh * 16:(hh + 1) * 16] = jnp.zeros((16,),
                                                              jnp.float32)
            return carry

        lax.fori_loop(0, CHUNK, zbody, 0)
        row0 = s * RPT
        off = 0
        for sz in (128, 128, 128, 128, 120):
            pltpu.sync_copy(bufs[0].at[pl.ds(0, sz)],
                            acc_sh.at[pl.ds(row0 + off, sz)])
            off += sz
        if stage_table:
            pltpu.sync_copy(v_hbm.at[pl.ds(row0, RPT)],
                            table.at[pl.ds(row0, RPT)])

        pltpu.sync_copy(srcw_hbm.at[wid], src_v)
        pltpu.sync_copy(dstw_hbm.at[wid], dst_v)
        plsc.subcore_barrier()
        for b in range(nbuf):
            pltpu.async_copy(table.at[src_v.at[b]], bufs[b], gsems[b])

        def body(g, carry):
            for b in range(nbuf):
                j = g * nbuf + b
                pltpu.make_async_copy(table.at[src_v.at[j]],
                                      bufs[b], gsems[b]).wait()
                pltpu.async_copy(bufs[b], acc_sh.at[dst_v.at[j]], ssems[b],
                                 add=True)
            for b in range(nbuf):
                j = g * nbuf + b
                pltpu.make_async_copy(bufs[b], acc_sh.at[dst_v.at[j]],
                                      ssems[b]).wait()
                nj = j + nbuf

                @pl.when(nj < CPT)
                def _():
                    pltpu.async_copy(table.at[src_v.at[nj]],
                                     bufs[b], gsems[b])
            return carry

        lax.fori_loop(0, CPT // nbuf, body, 0)
        plsc.subcore_barrier()
        pltpu.sync_copy(acc_sh.at[pl.ds(row0, RPT)],
                        out_hbm.at[c, pl.ds(row0, RPT)])

    return agg


_agg32 = _make_agg(H, 8, True)
_agg64 = _make_agg(D // 2, 4, False)


def _xterm_kernel(o1_ref, ag_ref, w_ref, o_ref):
    xa = o1_ref[...] + ag_ref[0] + ag_ref[1]
    o_ref[...] = jnp.dot(xa, w_ref[...], preferred_element_type=jnp.float32)


def _xterm1_kernel(x_ref, aglo_ref, aghi_ref, w_ref, o_ref):
    hw = D // 2
    xlo = x_ref[:, :hw] + aglo_ref[0] + aglo_ref[1]
    xhi = x_ref[:, hw:] + aghi_ref[0] + aghi_ref[1]
    o_ref[...] = (jnp.dot(xlo, w_ref[:hw],
                          preferred_element_type=jnp.float32) +
                  jnp.dot(xhi, w_ref[hw:],
                          preferred_element_type=jnp.float32))


def _gate_a_kernel(xterm_ref, h_ref, aggh_ref, whzr_ref, bzr_ref,
                   z_ref, g_ref, xc_ref):
    h = h_ref[...]
    ha = h + aggh_ref[0] + aggh_ref[1]
    hz = jnp.dot(ha, whzr_ref[...],
                 preferred_element_type=jnp.float32) + bzr_ref[...]
    xt = xterm_ref[...]
    z = jax.nn.sigmoid(xt[:, :H] + hz[:, :H])
    r = jax.nn.sigmoid(xt[:, H:2 * H] + hz[:, H:])
    z_ref[...] = z
    g_ref[...] = r * h
    xc_ref[...] = xt[:, 2 * H:]


def _gate_b_kernel(z_ref, h_ref, g_ref, xc_ref, aggg_ref, whh_ref, bh_ref,
                   hn_ref):
    ga = g_ref[...] + aggg_ref[0] + aggg_ref[1]
    c = jnp.tanh(xc_ref[...] +
                 jnp.dot(ga, whh_ref[...], preferred_element_type=jnp.float32)
                 + bh_ref[...])
    z = z_ref[...]
    hn_ref[...] = z * h_ref[...] + (1.0 - z) * c


def _head_kernel(h_ref, w_ref, b_ref, o_ref):
    o_ref[...] = jax.nn.relu(
        jnp.dot(h_ref[...], w_ref[...], preferred_element_type=jnp.float32)
        + b_ref[...])


def _tc(body, out_shape, grid=None, in_specs=None, out_specs=None):
    kwargs = {}
    if grid is not None:
        kwargs = dict(grid=grid, in_specs=in_specs, out_specs=out_specs)
    return pl.pallas_call(body, out_shape=out_shape, **kwargs)


def kernel(x, edge_index, params):
    p1, p2 = params['l1'], params['l2']

    Wx1 = jnp.concatenate([p1['Wxz'], p1['Wxr'], p1['Wxh']], axis=1)
    Wx2 = jnp.concatenate([p2['Wxz'], p2['Wxr'], p2['Wxh']], axis=1)
    Whzr1 = jnp.concatenate([p1['Whz'], p1['Whr']], axis=1)
    Whzr2 = jnp.concatenate([p2['Whz'], p2['Whr']], axis=1)
    bzr1 = jnp.concatenate([p1['bz'], p1['br']]).reshape(1, 2 * H)
    bzr2 = jnp.concatenate([p2['bz'], p2['br']]).reshape(1, 2 * H)
    bh1 = p1['bh'].reshape(1, H)
    bh2 = p2['bh'].reshape(1, H)
    bfc = params['bfc'].reshape(1, P * OUT)

    pad = EP - E
    srcp = jnp.concatenate([edge_index[0],
                            jnp.zeros((pad,), jnp.int32)])
    dstp = jnp.concatenate([edge_index[1],
                            jnp.full((pad,), NP - 1, jnp.int32)])
    srcw = srcp.reshape(NW, CPT, CHUNK)
    dstw = dstp.reshape(NW, CPT, CHUNK)

    x_pad = jnp.pad(x, ((0, 0), (0, NP - N), (0, 0)))

    def xterm_tc(v_t, ag_t, Wx):
        return _tc(
            _xterm_kernel,
            jax.ShapeDtypeStruct((NP, 3 * H), jnp.float32),
        )(v_t, ag_t, Wx)

    def gate_a(xterm_t, h, aggh, Whzr, bzr):
        return _tc(
            _gate_a_kernel,
            (jax.ShapeDtypeStruct((NP, H), jnp.float32),
             jax.ShapeDtypeStruct((NP, H), jnp.float32),
             jax.ShapeDtypeStruct((NP, H), jnp.float32)),
            grid=(NB,),
            in_specs=[pl.BlockSpec((BR, 3 * H), lambda i: (i, 0)),
                      pl.BlockSpec((BR, H), lambda i: (i, 0)),
                      pl.BlockSpec((NC, BR, H), lambda i: (0, i, 0)),
                      pl.BlockSpec((H, 2 * H), lambda i: (0, 0)),
                      pl.BlockSpec((1, 2 * H), lambda i: (0, 0))],
            out_specs=[pl.BlockSpec((BR, H), lambda i: (i, 0))] * 3,
        )(xterm_t, h, aggh, Whzr, bzr)

    def gate_b(z, h, g, xc, aggg, Whh, bh):
        return _tc(
            _gate_b_kernel,
            jax.ShapeDtypeStruct((NP, H), jnp.float32),
            grid=(NB,),
            in_specs=[pl.BlockSpec((BR, H), lambda i: (i, 0)),
                      pl.BlockSpec((BR, H), lambda i: (i, 0)),
                      pl.BlockSpec((BR, H), lambda i: (i, 0)),
                      pl.BlockSpec((BR, H), lambda i: (i, 0)),
                      pl.BlockSpec((NC, BR, H), lambda i: (0, i, 0)),
                      pl.BlockSpec((H, H), lambda i: (0, 0)),
                      pl.BlockSpec((1, H), lambda i: (0, 0))],
            out_specs=pl.BlockSpec((BR, H), lambda i: (i, 0)),
        )(z, h, g, xc, aggg, Whh, bh)

    xterm1 = []
    for t in range(T):
        aglo_t = _agg64(x_pad[t, :, :D // 2], srcw, dstw)
        aghi_t = _agg64(x_pad[t, :, D // 2:], srcw, dstw)
        xterm1.append(_tc(
            _xterm1_kernel,
            jax.ShapeDtypeStruct((NP, 3 * H), jnp.float32),
        )(x_pad[t], aglo_t, aghi_t, Wx1))

    def run_layer(xterms, h, Whzr, bzr, Whh, bh):
        outs = []
        for t in range(T):
            aggh = _agg32(h, srcw, dstw)
            z, g, xc = gate_a(xterms[t], h, aggh, Whzr, bzr)
            aggg = _agg32(g, srcw, dstw)
            h = gate_b(z, h, g, xc, aggg, Whh, bh)
            outs.append(h)
        return h, outs

    h0 = jnp.zeros((NP, H), jnp.float32)
    h1, out1 = run_layer(xterm1, h0, Whzr1, bzr1, p1['Whh'], bh1)

    xterm2 = []
    for t in range(T):
        ago_t = _agg32(out1[t], srcw, dstw)
        xterm2.append(xterm_tc(out1[t], ago_t, Wx2))

    h2, _ = run_layer(xterm2, h1, Whzr2, bzr2, p2['Whh'], bh2)

    y = _tc(
        _head_kernel,
        jax.ShapeDtypeStruct((NP, P * OUT), jnp.float32),
    )(h2, params['Wfc'], bfc)

    y = y[:N].reshape(N, P, OUT)
    return jnp.swapaxes(y, 0, 1)
